# baseline (speedup 1.0000x reference)
import functools

import jax
import jax.numpy as jnp
from jax import lax
from jax.experimental import pallas as pl
from jax.experimental.pallas import tpu as pltpu
from jax.experimental.pallas import tpu_sc as plsc

_GRID = 2


def _x_mask_kernel(x_ref, m_ref, xo_ref, mo_ref):
    xo_ref[...] = x_ref[...]

    @pl.when(pl.program_id(0) == 0)
    def _():
        mo_ref[...] = m_ref[...] != 0.0


def _sc_copy_body(chunk, v_hbm, out_hbm, scratch):
    info = plsc.get_sparse_core_info()
    wid = lax.axis_index("s") * info.num_cores + lax.axis_index("c")
    base = wid * chunk
    pltpu.sync_copy(v_hbm.at[pl.ds(base, chunk)], scratch)
    pltpu.sync_copy(scratch, out_hbm.at[pl.ds(base, chunk)])


def _sc_copy(vflat):
    info = plsc.get_sparse_core_info()
    nw = info.num_cores * info.num_subcores
    chunk = vflat.shape[0] // nw
    mesh = plsc.VectorSubcoreMesh(core_axis_name="c", subcore_axis_name="s")
    body = functools.partial(_sc_copy_body, chunk)
    return pl.kernel(
        body,
        mesh=mesh,
        out_type=jax.ShapeDtypeStruct(vflat.shape, vflat.dtype),
        scratch_types=[pltpu.VMEM((chunk,), vflat.dtype)],
    )(vflat)


def kernel(x, v, mask):
    b, n, l = x.shape
    _, nm, _ = mask.shape
    rows = b * n
    blk = rows // _GRID
    x2 = x.reshape(rows, l)
    xo, mo = pl.pallas_call(
        _x_mask_kernel,
        grid=(_GRID,),
        in_specs=[
            pl.BlockSpec((blk, l), lambda i: (i, 0)),
            pl.BlockSpec((b, nm, l), lambda i: (0, 0, 0)),
        ],
        out_specs=[
            pl.BlockSpec((blk, l), lambda i: (i, 0)),
            pl.BlockSpec((b, nm, l), lambda i: (0, 0, 0)),
        ],
        out_shape=[
            jax.ShapeDtypeStruct((rows, l), x.dtype),
            jax.ShapeDtypeStruct(mask.shape, jnp.bool_),
        ],
    )(x2, mask)
    vo = _sc_copy(v.reshape(-1)).reshape(v.shape)
    return (xo.reshape(x.shape), vo, mo)


# speedup vs baseline: 2.4218x; 2.4218x over previous
import jax
import jax.numpy as jnp
from jax.experimental import pallas as pl

_GRID = 2


def _trim_kernel(x_ref, v_ref, m_ref, xo_ref, vo_ref, mo_ref):
    xo_ref[...] = x_ref[...]

    @pl.when(pl.program_id(0) == 0)
    def _():
        vo_ref[...] = v_ref[...]
        mo_ref[...] = m_ref[...] != 0.0


def kernel(x, v, mask):
    b, n, l = x.shape
    _, nv, _ = v.shape
    _, nm, _ = mask.shape
    rows = b * n
    blk = rows // _GRID
    x2 = x.reshape(rows, l)
    xo, vo, mo = pl.pallas_call(
        _trim_kernel,
        grid=(_GRID,),
        in_specs=[
            pl.BlockSpec((blk, l), lambda i: (i, 0)),
            pl.BlockSpec((b, nv, l), lambda i: (0, 0, 0)),
            pl.BlockSpec((b, nm, l), lambda i: (0, 0, 0)),
        ],
        out_specs=[
            pl.BlockSpec((blk, l), lambda i: (i, 0)),
            pl.BlockSpec((b, nv, l), lambda i: (0, 0, 0)),
            pl.BlockSpec((b, nm, l), lambda i: (0, 0, 0)),
        ],
        out_shape=[
            jax.ShapeDtypeStruct((rows, l), x.dtype),
            jax.ShapeDtypeStruct(v.shape, v.dtype),
            jax.ShapeDtypeStruct(mask.shape, jnp.bool_),
        ],
    )(x2, v, mask)
    return (xo.reshape(x.shape), vo, mo)
